# baseline (speedup 1.0000x reference)
import functools

import jax
import jax.numpy as jnp
from jax import lax
from jax.experimental import pallas as pl
from jax.experimental.pallas import tpu as pltpu
from jax.experimental.pallas import tpu_sc as plsc

N = 10000
F = 128
BM = 400
NB = N // BM

LANES = 16
FLAT = N * N
NPAD = 10240
NW = 32
BPW = NPAD // NW
CHUNKS = BPW // LANES

_mesh = plsc.VectorSubcoreMesh(core_axis_name="c", subcore_axis_name="s")


@functools.partial(
    pl.kernel,
    mesh=_mesh,
    out_type=jax.ShapeDtypeStruct((NPAD,), jnp.float32),
    scratch_types=[
        pltpu.VMEM((BPW,), jnp.float32),
        pltpu.SemaphoreType.DMA,
    ],
)
def _sc_diag(adj1_hbm, d_hbm, d_v, sem):
    wid = lax.axis_index("s") * 2 + lax.axis_index("c")
    base = wid * BPW
    iota = lax.iota(jnp.int32, LANES)

    copies = []
    for c in range(CHUNKS):
        gi = base + c * LANES + iota
        flat = jnp.minimum(gi * (N + 1), FLAT - 1)
        copies.append(
            pltpu.async_copy(
                adj1_hbm.at[flat], d_v.at[pl.ds(c * LANES, LANES)], sem
            )
        )
    for cp in copies:
        cp.wait()

    pltpu.sync_copy(d_v, d_hbm.at[pl.ds(base, BPW)])


def _prep_kernel(x_ref, w_ref, sup16_ref):
    s = jnp.dot(x_ref[...], w_ref[...], preferred_element_type=jnp.float32)
    sup16_ref[...] = s.astype(jnp.bfloat16)


def _main_kernel(adj_ref, sup16_ref, d_ref, b_ref, out_ref, r_ref):
    i = pl.program_id(0)

    @pl.when(i == 0)
    def _():
        r_ref[...] = jnp.dot(
            d_ref[...], sup16_ref[...], preferred_element_type=jnp.float32
        )

    a16 = adj_ref[...].astype(jnp.bfloat16)
    y = jnp.dot(a16, sup16_ref[...], preferred_element_type=jnp.float32)
    out_ref[...] = y + r_ref[...] + b_ref[...]


@jax.jit
def kernel(input, adj, W, b):
    b2 = b.reshape(1, F)

    adj1 = adj.reshape(FLAT)
    d = _sc_diag(adj1)
    d16 = d[:N].astype(jnp.bfloat16).reshape(1, N)

    sup16 = pl.pallas_call(
        _prep_kernel,
        in_specs=[
            pl.BlockSpec((N, F), lambda: (0, 0)),
            pl.BlockSpec((F, F), lambda: (0, 0)),
        ],
        out_specs=pl.BlockSpec((N, F), lambda: (0, 0)),
        out_shape=jax.ShapeDtypeStruct((N, F), jnp.bfloat16),
    )(input, W)

    out = pl.pallas_call(
        _main_kernel,
        grid=(NB,),
        in_specs=[
            pl.BlockSpec((BM, N), lambda i: (i, 0)),
            pl.BlockSpec((N, F), lambda i: (0, 0)),
            pl.BlockSpec((1, N), lambda i: (0, 0)),
            pl.BlockSpec((1, F), lambda i: (0, 0)),
        ],
        out_specs=pl.BlockSpec((BM, F), lambda i: (i, 0)),
        out_shape=jax.ShapeDtypeStruct((N, F), jnp.float32),
        scratch_shapes=[pltpu.VMEM((1, F), jnp.float32)],
    )(adj, sup16, d16, b2)
    return out


# speedup vs baseline: 3.0957x; 3.0957x over previous
import jax
import jax.numpy as jnp
from jax.experimental import pallas as pl
from jax.experimental.pallas import tpu as pltpu

N = 10000
F = 128
BM = 400
NB = N // BM

DB = 128
ND = (N + DB - 1) // DB
NPAD = ND * DB


def _diag_kernel(adj_ref, d_ref):
    a = adj_ref[...]
    ri = jax.lax.broadcasted_iota(jnp.int32, (DB, DB), 0)
    ci = jax.lax.broadcasted_iota(jnp.int32, (DB, DB), 1)
    d_ref[...] = jnp.sum(jnp.where(ri == ci, a, 0.0), axis=0, keepdims=True)


def _main_kernel(adj_ref, x_ref, w_ref, d_ref, b_ref, out_ref, s16_ref, r_ref):
    i = pl.program_id(0)

    @pl.when(i == 0)
    def _():
        s = jnp.dot(x_ref[...], w_ref[...], preferred_element_type=jnp.float32)
        d = d_ref[...][:, :N]
        r_ref[...] = jnp.dot(d, s, preferred_element_type=jnp.float32)
        s16_ref[...] = s.astype(jnp.bfloat16)

    a16 = adj_ref[...].astype(jnp.bfloat16)
    y = jnp.dot(a16, s16_ref[...], preferred_element_type=jnp.float32)
    out_ref[...] = y + r_ref[...] + b_ref[...]


@jax.jit
def kernel(input, adj, W, b):
    b2 = b.reshape(1, F)

    d = pl.pallas_call(
        _diag_kernel,
        grid=(ND,),
        in_specs=[pl.BlockSpec((DB, DB), lambda i: (i, i))],
        out_specs=pl.BlockSpec((1, DB), lambda i: (0, i)),
        out_shape=jax.ShapeDtypeStruct((1, NPAD), jnp.float32),
    )(adj)

    out = pl.pallas_call(
        _main_kernel,
        grid=(NB,),
        in_specs=[
            pl.BlockSpec((BM, N), lambda i: (i, 0)),
            pl.BlockSpec((N, F), lambda i: (0, 0)),
            pl.BlockSpec((F, F), lambda i: (0, 0)),
            pl.BlockSpec((1, NPAD), lambda i: (0, 0)),
            pl.BlockSpec((1, F), lambda i: (0, 0)),
        ],
        out_specs=pl.BlockSpec((BM, F), lambda i: (i, 0)),
        out_shape=jax.ShapeDtypeStruct((N, F), jnp.float32),
        scratch_shapes=[
            pltpu.VMEM((N, F), jnp.bfloat16),
            pltpu.VMEM((1, F), jnp.float32),
        ],
    )(adj, input, W, d, b2)
    return out


# speedup vs baseline: 3.3169x; 1.0715x over previous
import jax
import jax.numpy as jnp
from jax.experimental import pallas as pl
from jax.experimental.pallas import tpu as pltpu

N = 10000
F = 128
BM = 400
NB = N // BM

DB = 256
ND = (N + DB - 1) // DB
NPAD = ND * DB


def _pre_kernel(adj_ref, x_ref, w_ref, b_ref, s16_ref, rb_ref, rx_ref):
    i = pl.program_id(0)
    base = i * DB
    a = adj_ref[...]
    x = x_ref[...]
    w = w_ref[...]

    ri = jax.lax.broadcasted_iota(jnp.int32, (DB, DB), 0)
    ci = jax.lax.broadcasted_iota(jnp.int32, (DB, DB), 1)
    dmask = (ri == ci) & (ci + base < N)
    d_blk = jnp.sum(jnp.where(dmask, a, 0.0), axis=0, keepdims=True)

    xrow = jax.lax.broadcasted_iota(jnp.int32, (DB, F), 0)
    x_safe = jnp.where(xrow + base < N, x, 0.0)

    s = jnp.dot(x, w, preferred_element_type=jnp.float32)
    s16_ref[...] = s.astype(jnp.bfloat16)

    @pl.when(i == 0)
    def _():
        rx_ref[...] = jnp.zeros_like(rx_ref)

    rx_ref[...] += jnp.dot(d_blk, x_safe, preferred_element_type=jnp.float32)

    @pl.when(i == ND - 1)
    def _():
        rb_ref[...] = (
            jnp.dot(rx_ref[...], w, preferred_element_type=jnp.float32)
            + b_ref[...].reshape(1, F)
        )


def _main_kernel(adj_ref, s16_ref, rb_ref, out_ref):
    a16 = adj_ref[...].astype(jnp.bfloat16)
    y = jnp.dot(a16, s16_ref[...], preferred_element_type=jnp.float32)
    out_ref[...] = y + rb_ref[...]


@jax.jit
def kernel(input, adj, W, b):
    s16, rb = pl.pallas_call(
        _pre_kernel,
        grid=(ND,),
        in_specs=[
            pl.BlockSpec((DB, DB), lambda i: (i, i)),
            pl.BlockSpec((DB, F), lambda i: (i, 0)),
            pl.BlockSpec((F, F), lambda i: (0, 0)),
            pl.BlockSpec((F,), lambda i: (0,)),
        ],
        out_specs=[
            pl.BlockSpec((DB, F), lambda i: (i, 0)),
            pl.BlockSpec((1, F), lambda i: (0, 0)),
        ],
        out_shape=[
            jax.ShapeDtypeStruct((NPAD, F), jnp.bfloat16),
            jax.ShapeDtypeStruct((1, F), jnp.float32),
        ],
        scratch_shapes=[pltpu.VMEM((1, F), jnp.float32)],
    )(adj, input, W, b)

    out = pl.pallas_call(
        _main_kernel,
        grid=(NB,),
        in_specs=[
            pl.BlockSpec((BM, N), lambda i: (i, 0)),
            pl.BlockSpec((N, F), lambda i: (0, 0)),
            pl.BlockSpec((1, F), lambda i: (0, 0)),
        ],
        out_specs=pl.BlockSpec((BM, F), lambda i: (i, 0)),
        out_shape=jax.ShapeDtypeStruct((N, F), jnp.float32),
    )(adj, s16, rb)
    return out
